# Initial kernel scaffold; baseline (speedup 1.0000x reference)
import jax
import jax.numpy as jnp
from jax.experimental import pallas as pl


def kernel(image, adj_s, W1a, b1a, W1b, b1b, W2a, b2a, W2b, b2b, W3a, b3a, W3b, b3b, Waa, baa, Wab, bab, Wac, bac, Wr, br, Wcls, bcls):
    raise NotImplementedError("write your pallas kernel here")


# speedup vs baseline: 1.4859x; 1.4859x over previous
import functools

import jax
import jax.numpy as jnp
from jax.experimental import pallas as pl
from jax.experimental.pallas import tpu as pltpu

N = 10000
H = 256
C = 3

BK = 1000
BI = 1024


def _agg_kernel(adj_ref, x_ref, out_ref):
    k = pl.program_id(1)

    @pl.when(k == 0)
    def _():
        out_ref[...] = jnp.zeros_like(out_ref)

    out_ref[...] += jax.lax.dot_general(
        adj_ref[...], x_ref[...],
        dimension_numbers=(((0,), (0,)), ((), ())),
        precision=jax.lax.Precision.HIGHEST,
        preferred_element_type=jnp.float32,
    )


def _agg(adj, x):
    grid = (pl.cdiv(N, BI), N // BK)
    return pl.pallas_call(
        _agg_kernel,
        grid=grid,
        in_specs=[
            pl.BlockSpec((BK, BI), lambda i, k: (k, i)),
            pl.BlockSpec((BK, H), lambda i, k: (k, 0)),
        ],
        out_specs=pl.BlockSpec((BI, H), lambda i, k: (i, 0)),
        out_shape=jax.ShapeDtypeStruct((N, H), jnp.float32),
    )(adj, x)


def _mlp_kernel(x_ref, agg_ref, wa_ref, ba_ref, wb_ref, bb_ref, out_ref):
    h = x_ref[...] + agg_ref[...]
    h = jnp.maximum(
        jax.lax.dot_general(h, wa_ref[...], (((1,), (0,)), ((), ())),
                            precision=jax.lax.Precision.HIGHEST,
                            preferred_element_type=jnp.float32) + ba_ref[...],
        0.0)
    h = jax.lax.dot_general(h, wb_ref[...], (((1,), (0,)), ((), ())),
                            precision=jax.lax.Precision.HIGHEST,
                            preferred_element_type=jnp.float32) + bb_ref[...]
    out_ref[...] = jnp.maximum(h, 0.0)


def _mlp(x, agg, Wa, ba, Wb, bb):
    B = 2000
    grid = (N // B,)
    return pl.pallas_call(
        _mlp_kernel,
        grid=grid,
        in_specs=[
            pl.BlockSpec((B, H), lambda i: (i, 0)),
            pl.BlockSpec((B, H), lambda i: (i, 0)),
            pl.BlockSpec((H, H), lambda i: (0, 0)),
            pl.BlockSpec((1, H), lambda i: (0, 0)),
            pl.BlockSpec((H, H), lambda i: (0, 0)),
            pl.BlockSpec((1, H), lambda i: (0, 0)),
        ],
        out_specs=pl.BlockSpec((B, H), lambda i: (i, 0)),
        out_shape=jax.ShapeDtypeStruct((N, H), jnp.float32),
    )(x, agg, Wa, ba.reshape(1, H), Wb, bb.reshape(1, H))


def _tail_kernel(x_ref, waa_ref, baa_ref, wab_ref, bab_ref, wac_ref, bac_ref,
                 wr_ref, br_ref, wcls_ref, bcls_ref, out_ref,
                 m_ref, l_ref, acc_ref):
    i = pl.program_id(0)
    nb = pl.num_programs(0)

    @pl.when(i == 0)
    def _():
        m_ref[0, 0] = -jnp.inf
        l_ref[0, 0] = 0.0
        acc_ref[...] = jnp.zeros_like(acc_ref)

    xb = x_ref[...]
    a = jnp.tanh(jax.lax.dot_general(xb, waa_ref[...], (((1,), (0,)), ((), ())),
                                     precision=jax.lax.Precision.HIGHEST,
                                     preferred_element_type=jnp.float32)
                 + baa_ref[...])
    b = 1.0 / (1.0 + jnp.exp(-(jax.lax.dot_general(
        xb, wab_ref[...], (((1,), (0,)), ((), ())),
        precision=jax.lax.Precision.HIGHEST,
        preferred_element_type=jnp.float32) + bab_ref[...])))
    s = jax.lax.dot_general(a * b, wac_ref[...], (((1,), (0,)), ((), ())),
                            precision=jax.lax.Precision.HIGHEST,
                            preferred_element_type=jnp.float32) + bac_ref[0, 0]
    bm = jnp.max(s)
    m_old = m_ref[0, 0]
    m_new = jnp.maximum(m_old, bm)
    scale = jnp.exp(m_old - m_new)
    p = jnp.exp(s - m_new)
    l_ref[0, 0] = l_ref[0, 0] * scale + jnp.sum(p)
    acc_ref[...] = acc_ref[...] * scale + jax.lax.dot_general(
        p, xb, (((0,), (0,)), ((), ())),
        precision=jax.lax.Precision.HIGHEST,
        preferred_element_type=jnp.float32)
    m_ref[0, 0] = m_new

    @pl.when(i == nb - 1)
    def _():
        h = acc_ref[...] / l_ref[0, 0]
        h = jnp.maximum(
            jax.lax.dot_general(h, wr_ref[...], (((1,), (0,)), ((), ())),
                                precision=jax.lax.Precision.HIGHEST,
                                preferred_element_type=jnp.float32)
            + br_ref[...], 0.0)
        out_ref[...] = jax.lax.dot_general(
            h, wcls_ref[...], (((1,), (0,)), ((), ())),
            precision=jax.lax.Precision.HIGHEST,
            preferred_element_type=jnp.float32) + bcls_ref[...]


def _tail(x3, Waa, baa, Wab, bab, Wac, bac, Wr, br, Wcls, bcls):
    B = 2000
    grid = (N // B,)
    full = lambda i: (0, 0)
    out = pl.pallas_call(
        _tail_kernel,
        grid=grid,
        in_specs=[
            pl.BlockSpec((B, H), lambda i: (i, 0)),
            pl.BlockSpec((H, H), full),
            pl.BlockSpec((1, H), full),
            pl.BlockSpec((H, H), full),
            pl.BlockSpec((1, H), full),
            pl.BlockSpec((H, 1), full),
            pl.BlockSpec((1, 1), full),
            pl.BlockSpec((H, H), full),
            pl.BlockSpec((1, H), full),
            pl.BlockSpec((H, C), full),
            pl.BlockSpec((1, C), full),
        ],
        out_specs=pl.BlockSpec((1, C), full),
        out_shape=jax.ShapeDtypeStruct((1, C), jnp.float32),
        scratch_shapes=[
            pltpu.SMEM((1, 1), jnp.float32),
            pltpu.SMEM((1, 1), jnp.float32),
            pltpu.VMEM((1, H), jnp.float32),
        ],
    )(x3, Waa, baa.reshape(1, H), Wab, bab.reshape(1, H), Wac,
      bac.reshape(1, 1), Wr, br.reshape(1, H), Wcls, bcls.reshape(1, C))
    return out.reshape(C)


def kernel(image, adj_s, W1a, b1a, W1b, b1b, W2a, b2a, W2b, b2b, W3a, b3a,
           W3b, b3b, Waa, baa, Wab, bab, Wac, bac, Wr, br, Wcls, bcls):
    x1 = _mlp(image, _agg(adj_s, image), W1a, b1a, W1b, b1b)
    x2 = _mlp(x1, _agg(adj_s, x1), W2a, b2a, W2b, b2b)
    x3 = _mlp(x2, _agg(adj_s, x2), W3a, b3a, W3b, b3b)
    return _tail(x3, Waa, baa, Wab, bab, Wac, bac, Wr, br, Wcls, bcls)


# speedup vs baseline: 2.9076x; 1.9568x over previous
import functools

import jax
import jax.numpy as jnp
from jax.experimental import pallas as pl
from jax.experimental.pallas import tpu as pltpu

N = 10000
H = 256
C = 3

BK = 1000
BI = 1024


def _agg_kernel(adj_ref, x_ref, out_ref):
    k = pl.program_id(1)

    @pl.when(k == 0)
    def _():
        out_ref[...] = jnp.zeros_like(out_ref)

    out_ref[...] += jax.lax.dot_general(
        adj_ref[...], x_ref[...],
        dimension_numbers=(((0,), (0,)), ((), ())),
        precision=jax.lax.Precision.DEFAULT,
        preferred_element_type=jnp.float32,
    )


def _agg(adj, x):
    grid = (pl.cdiv(N, BI), N // BK)
    return pl.pallas_call(
        _agg_kernel,
        grid=grid,
        in_specs=[
            pl.BlockSpec((BK, BI), lambda i, k: (k, i)),
            pl.BlockSpec((BK, H), lambda i, k: (k, 0)),
        ],
        out_specs=pl.BlockSpec((BI, H), lambda i, k: (i, 0)),
        out_shape=jax.ShapeDtypeStruct((N, H), jnp.float32),
    )(adj, x)


def _mlp_kernel(x_ref, agg_ref, wa_ref, ba_ref, wb_ref, bb_ref, out_ref):
    h = x_ref[...] + agg_ref[...]
    h = jnp.maximum(
        jax.lax.dot_general(h, wa_ref[...], (((1,), (0,)), ((), ())),
                            precision=jax.lax.Precision.HIGHEST,
                            preferred_element_type=jnp.float32) + ba_ref[...],
        0.0)
    h = jax.lax.dot_general(h, wb_ref[...], (((1,), (0,)), ((), ())),
                            precision=jax.lax.Precision.HIGHEST,
                            preferred_element_type=jnp.float32) + bb_ref[...]
    out_ref[...] = jnp.maximum(h, 0.0)


def _mlp(x, agg, Wa, ba, Wb, bb):
    B = 2000
    grid = (N // B,)
    return pl.pallas_call(
        _mlp_kernel,
        grid=grid,
        in_specs=[
            pl.BlockSpec((B, H), lambda i: (i, 0)),
            pl.BlockSpec((B, H), lambda i: (i, 0)),
            pl.BlockSpec((H, H), lambda i: (0, 0)),
            pl.BlockSpec((1, H), lambda i: (0, 0)),
            pl.BlockSpec((H, H), lambda i: (0, 0)),
            pl.BlockSpec((1, H), lambda i: (0, 0)),
        ],
        out_specs=pl.BlockSpec((B, H), lambda i: (i, 0)),
        out_shape=jax.ShapeDtypeStruct((N, H), jnp.float32),
    )(x, agg, Wa, ba.reshape(1, H), Wb, bb.reshape(1, H))


def _tail_kernel(x_ref, waa_ref, baa_ref, wab_ref, bab_ref, wac_ref, bac_ref,
                 wr_ref, br_ref, wcls_ref, bcls_ref, out_ref,
                 m_ref, l_ref, acc_ref):
    i = pl.program_id(0)
    nb = pl.num_programs(0)

    @pl.when(i == 0)
    def _():
        m_ref[0, 0] = -jnp.inf
        l_ref[0, 0] = 0.0
        acc_ref[...] = jnp.zeros_like(acc_ref)

    xb = x_ref[...]
    a = jnp.tanh(jax.lax.dot_general(xb, waa_ref[...], (((1,), (0,)), ((), ())),
                                     precision=jax.lax.Precision.HIGHEST,
                                     preferred_element_type=jnp.float32)
                 + baa_ref[...])
    b = 1.0 / (1.0 + jnp.exp(-(jax.lax.dot_general(
        xb, wab_ref[...], (((1,), (0,)), ((), ())),
        precision=jax.lax.Precision.HIGHEST,
        preferred_element_type=jnp.float32) + bab_ref[...])))
    s = jax.lax.dot_general(a * b, wac_ref[...], (((1,), (0,)), ((), ())),
                            precision=jax.lax.Precision.HIGHEST,
                            preferred_element_type=jnp.float32) + bac_ref[0, 0]
    bm = jnp.max(s)
    m_old = m_ref[0, 0]
    m_new = jnp.maximum(m_old, bm)
    scale = jnp.exp(m_old - m_new)
    p = jnp.exp(s - m_new)
    l_ref[0, 0] = l_ref[0, 0] * scale + jnp.sum(p)
    acc_ref[...] = acc_ref[...] * scale + jax.lax.dot_general(
        p, xb, (((0,), (0,)), ((), ())),
        precision=jax.lax.Precision.HIGHEST,
        preferred_element_type=jnp.float32)
    m_ref[0, 0] = m_new

    @pl.when(i == nb - 1)
    def _():
        h = acc_ref[...] / l_ref[0, 0]
        h = jnp.maximum(
            jax.lax.dot_general(h, wr_ref[...], (((1,), (0,)), ((), ())),
                                precision=jax.lax.Precision.HIGHEST,
                                preferred_element_type=jnp.float32)
            + br_ref[...], 0.0)
        out_ref[...] = jax.lax.dot_general(
            h, wcls_ref[...], (((1,), (0,)), ((), ())),
            precision=jax.lax.Precision.HIGHEST,
            preferred_element_type=jnp.float32) + bcls_ref[...]


def _tail(x3, Waa, baa, Wab, bab, Wac, bac, Wr, br, Wcls, bcls):
    B = 2000
    grid = (N // B,)
    full = lambda i: (0, 0)
    out = pl.pallas_call(
        _tail_kernel,
        grid=grid,
        in_specs=[
            pl.BlockSpec((B, H), lambda i: (i, 0)),
            pl.BlockSpec((H, H), full),
            pl.BlockSpec((1, H), full),
            pl.BlockSpec((H, H), full),
            pl.BlockSpec((1, H), full),
            pl.BlockSpec((H, 1), full),
            pl.BlockSpec((1, 1), full),
            pl.BlockSpec((H, H), full),
            pl.BlockSpec((1, H), full),
            pl.BlockSpec((H, C), full),
            pl.BlockSpec((1, C), full),
        ],
        out_specs=pl.BlockSpec((1, C), full),
        out_shape=jax.ShapeDtypeStruct((1, C), jnp.float32),
        scratch_shapes=[
            pltpu.SMEM((1, 1), jnp.float32),
            pltpu.SMEM((1, 1), jnp.float32),
            pltpu.VMEM((1, H), jnp.float32),
        ],
    )(x3, Waa, baa.reshape(1, H), Wab, bab.reshape(1, H), Wac,
      bac.reshape(1, 1), Wr, br.reshape(1, H), Wcls, bcls.reshape(1, C))
    return out.reshape(C)


def kernel(image, adj_s, W1a, b1a, W1b, b1b, W2a, b2a, W2b, b2b, W3a, b3a,
           W3b, b3b, Waa, baa, Wab, bab, Wac, bac, Wr, br, Wcls, bcls):
    x1 = _mlp(image, _agg(adj_s, image), W1a, b1a, W1b, b1b)
    x2 = _mlp(x1, _agg(adj_s, x1), W2a, b2a, W2b, b2b)
    x3 = _mlp(x2, _agg(adj_s, x2), W3a, b3a, W3b, b3b)
    return _tail(x3, Waa, baa, Wab, bab, Wac, bac, Wr, br, Wcls, bcls)
